# baseline (speedup 1.0000x reference)
import functools

import jax
import jax.numpy as jnp
from jax import lax
from jax.experimental import pallas as pl
from jax.experimental.pallas import tpu as pltpu
from jax.experimental.pallas import tpu_sc as plsc

_N = 10000
_D = 128
_E = 320000
_NC = 2
_NS = 16
_NW = _NC * _NS
_EPT = _E // _NW
_K = 128
_CH = -(-_EPT // _K)
_EPTP = _CH * _K
_NH = 10240
_NPT = 632
_NPL = _N - (_NS - 1) * _NPT
_BM = 400


def _gemm_body(x_ref, w_ref, o_ref):
    o_ref[...] = lax.dot_general(
        x_ref[...], w_ref[...], (((1,), (1,)), ((), ())),
        preferred_element_type=jnp.float32)


def _add_body(a_ref, b_ref, o_ref):
    o_ref[...] = a_ref[...] + b_ref[...]


def _seg_body(src_hbm, dst_hbm, h_hbm, z_hbm, out_hbm,
              src_idx, dst_buf, rows, sems, dsems, acc):
    c = lax.axis_index("c")
    s = lax.axis_index("s")
    w = c * _NS + s
    pltpu.sync_copy(src_hbm.at[w], src_idx)
    @pl.when(s < _NS - 1)
    def _():
        pltpu.sync_copy(z_hbm.at[pl.ds(s * _NPT, _NPT)],
                        acc.at[pl.ds(s * _NPT, _NPT)])

    @pl.when(s == _NS - 1)
    def _():
        pltpu.sync_copy(z_hbm.at[pl.ds(s * _NPT, _NPL)],
                        acc.at[pl.ds(s * _NPT, _NPL)])

    plsc.subcore_barrier()

    def fire(j, b):
        pltpu.async_copy(dst_hbm.at[w, j], dst_buf.at[b], dsems.at[b])
        pltpu.async_copy(h_hbm.at[src_idx.at[pl.ds(j * _K, _K)]],
                         rows.at[b], sems.at[b])

    fire(0, 0)
    fire(1, 1)

    def chunk(j, carry):
        b = lax.rem(j, 2)
        pltpu.make_async_copy(dst_hbm.at[w, 0], dst_buf.at[b],
                              dsems.at[b]).wait()
        pltpu.make_async_copy(h_hbm.at[pl.ds(0, _K)], rows.at[b],
                              sems.at[b]).wait()
        pltpu.sync_copy(rows.at[b], acc.at[dst_buf.at[b]], add=True)

        @pl.when(j + 2 < _CH)
        def _():
            fire(j + 2, b)
        return carry

    lax.fori_loop(0, _CH, chunk, 0)

    plsc.subcore_barrier()

    @pl.when(s < _NS - 1)
    def _():
        pltpu.sync_copy(acc.at[pl.ds(s * _NPT, _NPT)],
                        out_hbm.at[c, pl.ds(s * _NPT, _NPT)])

    @pl.when(s == _NS - 1)
    def _():
        pltpu.sync_copy(acc.at[pl.ds(s * _NPT, _NPL)],
                        out_hbm.at[c, pl.ds(s * _NPT, _NPL)])


def kernel(x, edge_index, weight):
    n, d = x.shape

    x_pad = jnp.pad(x, ((0, _NH - n), (0, 0)))
    h = pl.pallas_call(
        _gemm_body,
        grid=(_NH // _BM,),
        in_specs=[pl.BlockSpec((_BM, d), lambda i: (i, 0)),
                  pl.BlockSpec(weight.shape, lambda i: (0, 0))],
        out_specs=pl.BlockSpec((_BM, d), lambda i: (i, 0)),
        out_shape=jax.ShapeDtypeStruct((_NH, d), jnp.float32),
    )(x_pad, weight)

    src = jnp.pad(edge_index[0].reshape(_NW, _EPT),
                  ((0, 0), (0, _EPTP - _EPT)), constant_values=n)
    dst = jnp.pad(edge_index[1].reshape(_NW, _EPT),
                  ((0, 0), (0, _EPTP - _EPT)),
                  constant_values=0).reshape(_NW, _CH, _K)
    zeros = jnp.zeros((n, d), jnp.float32)

    mesh = plsc.VectorSubcoreMesh(core_axis_name="c", subcore_axis_name="s")
    seg = pl.kernel(
        _seg_body,
        out_type=jax.ShapeDtypeStruct((_NC, n, d), jnp.float32),
        mesh=mesh,
        scratch_types=[
            pltpu.VMEM((_EPTP,), jnp.int32),
            pltpu.VMEM((2, _K), jnp.int32),
            pltpu.VMEM((2, _K, _D), jnp.float32),
            pltpu.SemaphoreType.DMA((2,)),
            pltpu.SemaphoreType.DMA((2,)),
            pltpu.VMEM_SHARED((_N, _D), jnp.float32),
        ],
    )
    parts = seg(src, dst, h, zeros)

    out = pl.pallas_call(
        _add_body,
        grid=(n // _BM,),
        in_specs=[pl.BlockSpec((_BM, d), lambda i: (i, 0)),
                  pl.BlockSpec((_BM, d), lambda i: (i, 0))],
        out_specs=pl.BlockSpec((_BM, d), lambda i: (i, 0)),
        out_shape=jax.ShapeDtypeStruct((n, d), jnp.float32),
    )(parts[0], parts[1])
    return out


# speedup vs baseline: 1.6366x; 1.6366x over previous
import functools

import jax
import jax.numpy as jnp
from jax import lax
from jax.experimental import pallas as pl
from jax.experimental.pallas import tpu as pltpu
from jax.experimental.pallas import tpu_sc as plsc

_N = 10000
_D = 128
_E = 320000
_NC = 2
_NS = 16
_NW = _NC * _NS
_EPT = _E // _NW
_K = 80
_CH = _EPT // _K
_NPT = 632
_NPL = _N - (_NS - 1) * _NPT
_BM = 400


def _gemm_body(x_ref, w_ref, o_ref):
    o_ref[...] = lax.dot_general(
        x_ref[...], w_ref[...], (((1,), (1,)), ((), ())),
        preferred_element_type=jnp.float32)


def _add_body(a_ref, b_ref, o_ref):
    o_ref[...] = a_ref[...] + b_ref[...]


def _seg_body(src_hbm, dst_hbm, h_hbm, z_hbm, out_hbm,
              src_idx, dst_idx, rows, sems, ssems, acc):
    c = lax.axis_index("c")
    s = lax.axis_index("s")
    w = c * _NS + s
    pltpu.sync_copy(src_hbm.at[w], src_idx)
    pltpu.sync_copy(dst_hbm.at[w], dst_idx)
    @pl.when(s < _NS - 1)
    def _():
        pltpu.sync_copy(z_hbm.at[pl.ds(s * _NPT, _NPT)],
                        acc.at[pl.ds(s * _NPT, _NPT)])

    @pl.when(s == _NS - 1)
    def _():
        pltpu.sync_copy(z_hbm.at[pl.ds(s * _NPT, _NPL)],
                        acc.at[pl.ds(s * _NPT, _NPL)])

    plsc.subcore_barrier()

    def drain(ref_sems, b):
        pltpu.make_async_copy(h_hbm.at[pl.ds(0, _K)], rows.at[b],
                              ref_sems.at[b]).wait()

    pltpu.async_copy(h_hbm.at[src_idx.at[pl.ds(0, _K)]],
                     rows.at[0], sems.at[0])

    def chunk(j, carry):
        b = lax.rem(j, 2)

        @pl.when(j > 0)
        def _():
            drain(ssems, 1 - b)

        @pl.when(j + 1 < _CH)
        def _():
            pltpu.async_copy(
                h_hbm.at[src_idx.at[pl.ds((j + 1) * _K, _K)]],
                rows.at[1 - b], sems.at[1 - b])

        drain(sems, b)
        pltpu.async_copy(rows.at[b], acc.at[dst_idx.at[j]],
                         ssems.at[b], add=True)
        return carry

    lax.fori_loop(0, _CH, chunk, 0)
    drain(ssems, (_CH - 1) % 2)

    plsc.subcore_barrier()

    @pl.when(s < _NS - 1)
    def _():
        pltpu.sync_copy(acc.at[pl.ds(s * _NPT, _NPT)],
                        out_hbm.at[c, pl.ds(s * _NPT, _NPT)])

    @pl.when(s == _NS - 1)
    def _():
        pltpu.sync_copy(acc.at[pl.ds(s * _NPT, _NPL)],
                        out_hbm.at[c, pl.ds(s * _NPT, _NPL)])


def kernel(x, edge_index, weight):
    n, d = x.shape

    h = pl.pallas_call(
        _gemm_body,
        grid=(n // _BM,),
        in_specs=[pl.BlockSpec((_BM, d), lambda i: (i, 0)),
                  pl.BlockSpec(weight.shape, lambda i: (0, 0))],
        out_specs=pl.BlockSpec((_BM, d), lambda i: (i, 0)),
        out_shape=jax.ShapeDtypeStruct((n, d), jnp.float32),
    )(x, weight)

    src = edge_index[0].reshape(_NW, _EPT)
    dst = edge_index[1].reshape(_NW, _CH, _K)
    zeros = jnp.zeros((n, d), jnp.float32)

    mesh = plsc.VectorSubcoreMesh(core_axis_name="c", subcore_axis_name="s")
    seg = pl.kernel(
        _seg_body,
        out_type=jax.ShapeDtypeStruct((_NC, n, d), jnp.float32),
        mesh=mesh,
        scratch_types=[
            pltpu.VMEM((_EPT,), jnp.int32),
            pltpu.VMEM((_CH, _K), jnp.int32),
            pltpu.VMEM((2, _K, _D), jnp.float32),
            pltpu.SemaphoreType.DMA((2,)),
            pltpu.SemaphoreType.DMA((2,)),
            pltpu.VMEM_SHARED((_N, _D), jnp.float32),
        ],
    )
    parts = seg(src, dst, h, zeros)

    out = pl.pallas_call(
        _add_body,
        grid=(n // _BM,),
        in_specs=[pl.BlockSpec((_BM, d), lambda i: (i, 0)),
                  pl.BlockSpec((_BM, d), lambda i: (i, 0))],
        out_specs=pl.BlockSpec((_BM, d), lambda i: (i, 0)),
        out_shape=jax.ShapeDtypeStruct((n, d), jnp.float32),
    )(parts[0], parts[1])
    return out


# speedup vs baseline: 1.8947x; 1.1577x over previous
import functools

import jax
import jax.numpy as jnp
from jax import lax
from jax.experimental import pallas as pl
from jax.experimental.pallas import tpu as pltpu
from jax.experimental.pallas import tpu_sc as plsc

_N = 10000
_D = 128
_E = 320000
_NC = 2
_NS = 16
_NW = _NC * _NS
_K = 64
_EPT = 10112
_EPL = _E - 31 * _EPT
_CH = _EPT // _K
_CHL = _EPL // _K
_NPT = 632
_NPL = _N - (_NS - 1) * _NPT
_BM = 2000


def _gemm_body(x_ref, w_ref, o_ref):
    o_ref[...] = lax.dot_general(
        x_ref[...], w_ref[...], (((1,), (1,)), ((), ())),
        preferred_element_type=jnp.float32)


def _add_body(a_ref, b_ref, o_ref):
    o_ref[...] = a_ref[...] + b_ref[...]


def _seg_body(e_hbm, h_hbm, z_hbm, out_hbm,
              eidx, rows, sems, ssems, acc):
    c = lax.axis_index("c")
    s = lax.axis_index("s")
    w = c * _NS + s
    nch = jnp.where(w < _NW - 1, _CH, _CHL)

    @pl.when(w < _NW - 1)
    def _():
        pltpu.sync_copy(e_hbm.at[:, pl.ds(w * _EPT, _EPT)], eidx)

    @pl.when(w == _NW - 1)
    def _():
        pltpu.sync_copy(e_hbm.at[:, pl.ds(w * _EPT, _EPL)],
                        eidx.at[:, pl.ds(0, _EPL)])
    @pl.when(s < _NS - 1)
    def _():
        pltpu.sync_copy(z_hbm.at[pl.ds(s * _NPT, _NPT)],
                        acc.at[pl.ds(s * _NPT, _NPT)])

    @pl.when(s == _NS - 1)
    def _():
        pltpu.sync_copy(z_hbm.at[pl.ds(s * _NPT, _NPL)],
                        acc.at[pl.ds(s * _NPT, _NPL)])

    plsc.subcore_barrier()

    def drain(ref_sems, b):
        pltpu.make_async_copy(h_hbm.at[pl.ds(0, _K)], rows.at[b],
                              ref_sems.at[b]).wait()

    pltpu.async_copy(h_hbm.at[eidx.at[0, pl.ds(0, _K)]],
                     rows.at[0], sems.at[0])

    def chunk(j, carry):
        b = lax.rem(j, 2)

        @pl.when(j > 0)
        def _():
            drain(ssems, 1 - b)

        @pl.when(j + 1 < nch)
        def _():
            pltpu.async_copy(
                h_hbm.at[eidx.at[0, pl.ds((j + 1) * _K, _K)]],
                rows.at[1 - b], sems.at[1 - b])

        drain(sems, b)
        pltpu.async_copy(rows.at[b],
                         acc.at[eidx.at[1, pl.ds(j * _K, _K)]],
                         ssems.at[b], add=True)
        return carry

    lax.fori_loop(0, nch, chunk, 0)
    drain(ssems, (_CH - 1) % 2)

    plsc.subcore_barrier()

    @pl.when(s < _NS - 1)
    def _():
        pltpu.sync_copy(acc.at[pl.ds(s * _NPT, _NPT)],
                        out_hbm.at[c, pl.ds(s * _NPT, _NPT)])

    @pl.when(s == _NS - 1)
    def _():
        pltpu.sync_copy(acc.at[pl.ds(s * _NPT, _NPL)],
                        out_hbm.at[c, pl.ds(s * _NPT, _NPL)])


def kernel(x, edge_index, weight):
    n, d = x.shape

    h = pl.pallas_call(
        _gemm_body,
        grid=(n // _BM,),
        in_specs=[pl.BlockSpec((_BM, d), lambda i: (i, 0)),
                  pl.BlockSpec(weight.shape, lambda i: (0, 0))],
        out_specs=pl.BlockSpec((_BM, d), lambda i: (i, 0)),
        out_shape=jax.ShapeDtypeStruct((n, d), jnp.float32),
    )(x, weight)

    zeros = jnp.zeros((n, d), jnp.float32)

    mesh = plsc.VectorSubcoreMesh(core_axis_name="c", subcore_axis_name="s")
    seg = pl.kernel(
        _seg_body,
        out_type=jax.ShapeDtypeStruct((_NC, n, d), jnp.float32),
        mesh=mesh,
        scratch_types=[
            pltpu.VMEM((2, _EPT), jnp.int32),
            pltpu.VMEM((2, _K, _D), jnp.float32),
            pltpu.SemaphoreType.DMA((2,)),
            pltpu.SemaphoreType.DMA((2,)),
            pltpu.VMEM_SHARED((_N, _D), jnp.float32),
        ],
    )
    parts = seg(edge_index, h, zeros)

    out = pl.pallas_call(
        _add_body,
        grid=(n // _BM,),
        in_specs=[pl.BlockSpec((_BM, d), lambda i: (i, 0)),
                  pl.BlockSpec((_BM, d), lambda i: (i, 0))],
        out_specs=pl.BlockSpec((_BM, d), lambda i: (i, 0)),
        out_shape=jax.ShapeDtypeStruct((n, d), jnp.float32),
    )(parts[0], parts[1])
    return out


# speedup vs baseline: 1.9886x; 1.0495x over previous
import functools

import jax
import jax.numpy as jnp
from jax import lax
from jax.experimental import pallas as pl
from jax.experimental.pallas import tpu as pltpu
from jax.experimental.pallas import tpu_sc as plsc

_N = 10000
_D = 128
_E = 320000
_NC = 2
_NS = 16
_NW = _NC * _NS
_K = 64
_EPT = 10112
_EPL = _E - 31 * _EPT
_CH = _EPT // _K
_CHL = _EPL // _K
_NPT = 632
_NPL = _N - (_NS - 1) * _NPT
_BM = 2000


def _gemm_body(x_ref, w_ref, o_ref, z_ref):
    o_ref[...] = lax.dot_general(
        x_ref[...], w_ref[...], (((1,), (1,)), ((), ())),
        preferred_element_type=jnp.float32)
    z_ref[...] = jnp.zeros_like(z_ref)


def _add_body(p_ref, o_ref):
    o_ref[...] = p_ref[0] + p_ref[1]


def _seg_body(e_hbm, h_hbm, z_hbm, out_hbm,
              eidx, rows, sems, ssems, acc):
    c = lax.axis_index("c")
    s = lax.axis_index("s")
    w = c * _NS + s
    nch = jnp.where(w < _NW - 1, _CH, _CHL)

    @pl.when(w < _NW - 1)
    def _():
        pltpu.sync_copy(e_hbm.at[:, pl.ds(w * _EPT, _EPT)], eidx)

    @pl.when(w == _NW - 1)
    def _():
        pltpu.sync_copy(e_hbm.at[:, pl.ds(w * _EPT, _EPL)],
                        eidx.at[:, pl.ds(0, _EPL)])
    @pl.when(s < _NS - 1)
    def _():
        pltpu.sync_copy(z_hbm.at[pl.ds(s * _NPT, _NPT)],
                        acc.at[pl.ds(s * _NPT, _NPT)])

    @pl.when(s == _NS - 1)
    def _():
        pltpu.sync_copy(z_hbm.at[pl.ds(s * _NPT, _NPL)],
                        acc.at[pl.ds(s * _NPT, _NPL)])

    plsc.subcore_barrier()

    def drain(ref_sems, b):
        pltpu.make_async_copy(h_hbm.at[pl.ds(0, _K)], rows.at[b],
                              ref_sems.at[b]).wait()

    pltpu.async_copy(h_hbm.at[eidx.at[0, pl.ds(0, _K)]],
                     rows.at[0], sems.at[0])

    def chunk(j, carry):
        b = lax.rem(j, 2)

        @pl.when(j > 0)
        def _():
            drain(ssems, 1 - b)

        @pl.when(j + 1 < nch)
        def _():
            pltpu.async_copy(
                h_hbm.at[eidx.at[0, pl.ds((j + 1) * _K, _K)]],
                rows.at[1 - b], sems.at[1 - b])

        drain(sems, b)
        pltpu.async_copy(rows.at[b],
                         acc.at[eidx.at[1, pl.ds(j * _K, _K)]],
                         ssems.at[b], add=True)
        return carry

    lax.fori_loop(0, nch, chunk, 0)
    drain(ssems, (_CH - 1) % 2)

    plsc.subcore_barrier()

    @pl.when(s < _NS - 1)
    def _():
        pltpu.sync_copy(acc.at[pl.ds(s * _NPT, _NPT)],
                        out_hbm.at[c, pl.ds(s * _NPT, _NPT)])

    @pl.when(s == _NS - 1)
    def _():
        pltpu.sync_copy(acc.at[pl.ds(s * _NPT, _NPL)],
                        out_hbm.at[c, pl.ds(s * _NPT, _NPL)])


def kernel(x, edge_index, weight):
    n, d = x.shape

    h, zeros = pl.pallas_call(
        _gemm_body,
        grid=(n // _BM,),
        in_specs=[pl.BlockSpec((_BM, d), lambda i: (i, 0)),
                  pl.BlockSpec(weight.shape, lambda i: (0, 0))],
        out_specs=[pl.BlockSpec((_BM, d), lambda i: (i, 0)),
                   pl.BlockSpec((_BM, d), lambda i: (i, 0))],
        out_shape=[jax.ShapeDtypeStruct((n, d), jnp.float32),
                   jax.ShapeDtypeStruct((n, d), jnp.float32)],
    )(x, weight)

    mesh = plsc.VectorSubcoreMesh(core_axis_name="c", subcore_axis_name="s")
    seg = pl.kernel(
        _seg_body,
        out_type=jax.ShapeDtypeStruct((_NC, n, d), jnp.float32),
        mesh=mesh,
        scratch_types=[
            pltpu.VMEM((2, _EPT), jnp.int32),
            pltpu.VMEM((3, _K, _D), jnp.float32),
            pltpu.SemaphoreType.DMA((3,)),
            pltpu.SemaphoreType.DMA((3,)),
            pltpu.VMEM_SHARED((_N, _D), jnp.float32),
        ],
    )
    parts = seg(edge_index, h, zeros)

    out = pl.pallas_call(
        _add_body,
        grid=(n // _BM,),
        in_specs=[pl.BlockSpec((_NC, _BM, d), lambda i: (0, i, 0))],
        out_specs=pl.BlockSpec((_BM, d), lambda i: (i, 0)),
        out_shape=jax.ShapeDtypeStruct((n, d), jnp.float32),
    )(parts)
    return out


# speedup vs baseline: 1.9888x; 1.0001x over previous
import functools

import jax
import jax.numpy as jnp
from jax import lax
from jax.experimental import pallas as pl
from jax.experimental.pallas import tpu as pltpu
from jax.experimental.pallas import tpu_sc as plsc

_N = 10000
_D = 128
_E = 320000
_NC = 2
_NS = 16
_NW = _NC * _NS
_K = 64
_EPT = 10112
_EPL = _E - 31 * _EPT
_CH = _EPT // _K
_CHL = _EPL // _K
_NPT = 632
_NPL = _N - (_NS - 1) * _NPT
_BM = 2000


def _gemm_body(x_ref, w_ref, o_ref, z_ref):
    o_ref[...] = lax.dot_general(
        x_ref[...], w_ref[...], (((1,), (1,)), ((), ())),
        preferred_element_type=jnp.float32)
    z_ref[...] = jnp.zeros_like(z_ref)


_NZ = 640


def _add_body(p_ref, o_ref):
    o_ref[...] = p_ref[0] + p_ref[1]


def _seg_body(e_hbm, h_hbm, z_hbm, out_hbm,
              eidx, rows, sems, ssems, acc):
    c = lax.axis_index("c")
    s = lax.axis_index("s")
    w = c * _NS + s
    nch = jnp.where(w < _NW - 1, _CH, _CHL)

    @pl.when(w < _NW - 1)
    def _():
        pltpu.sync_copy(e_hbm.at[:, pl.ds(w * _EPT, _EPT)], eidx)

    @pl.when(w == _NW - 1)
    def _():
        pltpu.sync_copy(e_hbm.at[:, pl.ds(w * _EPT, _EPL)],
                        eidx.at[:, pl.ds(0, _EPL)])
    @pl.when(s < _NS - 1)
    def _():
        pltpu.sync_copy(z_hbm.at[pl.ds(0, _NPT)],
                        acc.at[pl.ds(s * _NPT, _NPT)])

    @pl.when(s == _NS - 1)
    def _():
        pltpu.sync_copy(z_hbm.at[pl.ds(0, _NPL)],
                        acc.at[pl.ds(s * _NPT, _NPL)])

    plsc.subcore_barrier()

    def drain(ref_sems, b):
        pltpu.make_async_copy(h_hbm.at[pl.ds(0, _K)], rows.at[b],
                              ref_sems.at[b]).wait()

    pltpu.async_copy(h_hbm.at[eidx.at[0, pl.ds(0, _K)]],
                     rows.at[0], sems.at[0])

    def chunk(j, carry):
        b = lax.rem(j, 2)

        @pl.when(j > 0)
        def _():
            drain(ssems, 1 - b)

        @pl.when(j + 1 < nch)
        def _():
            pltpu.async_copy(
                h_hbm.at[eidx.at[0, pl.ds((j + 1) * _K, _K)]],
                rows.at[1 - b], sems.at[1 - b])

        drain(sems, b)
        pltpu.async_copy(rows.at[b],
                         acc.at[eidx.at[1, pl.ds(j * _K, _K)]],
                         ssems.at[b], add=True)
        return carry

    lax.fori_loop(0, nch, chunk, 0)
    drain(ssems, (_CH - 1) % 2)

    plsc.subcore_barrier()

    @pl.when(s < _NS - 1)
    def _():
        pltpu.sync_copy(acc.at[pl.ds(s * _NPT, _NPT)],
                        out_hbm.at[c, pl.ds(s * _NPT, _NPT)])

    @pl.when(s == _NS - 1)
    def _():
        pltpu.sync_copy(acc.at[pl.ds(s * _NPT, _NPL)],
                        out_hbm.at[c, pl.ds(s * _NPT, _NPL)])


def kernel(x, edge_index, weight):
    n, d = x.shape

    h, zeros = pl.pallas_call(
        _gemm_body,
        grid=(n // _BM,),
        in_specs=[pl.BlockSpec((_BM, d), lambda i: (i, 0)),
                  pl.BlockSpec(weight.shape, lambda i: (0, 0))],
        out_specs=[pl.BlockSpec((_BM, d), lambda i: (i, 0)),
                   pl.BlockSpec((_NZ, d), lambda i: (0, 0))],
        out_shape=[jax.ShapeDtypeStruct((n, d), jnp.float32),
                   jax.ShapeDtypeStruct((_NZ, d), jnp.float32)],
    )(x, weight)

    mesh = plsc.VectorSubcoreMesh(core_axis_name="c", subcore_axis_name="s")
    seg = pl.kernel(
        _seg_body,
        out_type=jax.ShapeDtypeStruct((_NC, n, d), jnp.float32),
        mesh=mesh,
        scratch_types=[
            pltpu.VMEM((2, _EPT), jnp.int32),
            pltpu.VMEM((3, _K, _D), jnp.float32),
            pltpu.SemaphoreType.DMA((3,)),
            pltpu.SemaphoreType.DMA((3,)),
            pltpu.VMEM_SHARED((_N, _D), jnp.float32),
        ],
    )
    parts = seg(edge_index, h, zeros)

    out = pl.pallas_call(
        _add_body,
        grid=(n // _BM,),
        in_specs=[pl.BlockSpec((_NC, _BM, d), lambda i: (0, i, 0))],
        out_specs=pl.BlockSpec((_BM, d), lambda i: (i, 0)),
        out_shape=jax.ShapeDtypeStruct((n, d), jnp.float32),
    )(parts)
    return out


# speedup vs baseline: 1.9948x; 1.0030x over previous
import jax
import jax.numpy as jnp
from jax import lax
from jax.experimental import pallas as pl
from jax.experimental.pallas import tpu as pltpu
from jax.experimental.pallas import tpu_sc as plsc

_N = 10000
_D = 128
_E = 320000
_NC = 2
_NS = 16
_NW = _NC * _NS
_K = 64
_EPT = 10112
_EPL = _E - 31 * _EPT
_CH = _EPT // _K
_CHL = _EPL // _K
_NPT = 632
_NPL = _N - (_NS - 1) * _NPT
_BM = 2000


def _gemm_body(x_ref, w_ref, o_ref, z_ref):
    o_ref[...] = lax.dot_general(
        x_ref[...], w_ref[...], (((1,), (1,)), ((), ())),
        preferred_element_type=jnp.float32)
    z_ref[...] = jnp.zeros_like(z_ref)


_NZ = 640


def _add_body(p_ref, o_ref):
    o_ref[...] = p_ref[0] + p_ref[1]


def _seg_body(e_hbm, h_hbm, z_hbm, out_hbm,
              eidx, rows, sems, ssems, acc):
    c = lax.axis_index("c")
    s = lax.axis_index("s")
    w = c * _NS + s
    nch = jnp.where(w < _NW - 1, _CH, _CHL)

    @pl.when(w < _NW - 1)
    def _():
        pltpu.sync_copy(e_hbm.at[:, pl.ds(w * _EPT, _EPT)], eidx)

    @pl.when(w == _NW - 1)
    def _():
        pltpu.sync_copy(e_hbm.at[:, pl.ds(w * _EPT, _EPL)],
                        eidx.at[:, pl.ds(0, _EPL)])
    @pl.when(s < _NS - 1)
    def _():
        pltpu.sync_copy(z_hbm.at[pl.ds(0, _NPT)],
                        acc.at[pl.ds(s * _NPT, _NPT)])

    @pl.when(s == _NS - 1)
    def _():
        pltpu.sync_copy(z_hbm.at[pl.ds(0, _NPL)],
                        acc.at[pl.ds(s * _NPT, _NPL)])

    plsc.subcore_barrier()

    def drain(ref_sems, b):
        pltpu.make_async_copy(h_hbm.at[pl.ds(0, _K)], rows.at[b],
                              ref_sems.at[b]).wait()

    pltpu.async_copy(h_hbm.at[eidx.at[0, pl.ds(0, _K)]],
                     rows.at[0], sems.at[0])

    def chunk(j, carry):
        b = lax.rem(j, 2)

        @pl.when(j > 0)
        def _():
            drain(ssems, 1 - b)

        @pl.when(j + 1 < nch)
        def _():
            pltpu.async_copy(
                h_hbm.at[eidx.at[0, pl.ds((j + 1) * _K, _K)]],
                rows.at[1 - b], sems.at[1 - b])

        drain(sems, b)
        pltpu.async_copy(rows.at[b],
                         acc.at[eidx.at[1, pl.ds(j * _K, _K)]],
                         ssems.at[b], add=True)
        return carry

    lax.fori_loop(0, nch, chunk, 0)
    drain(ssems, (_CH - 1) % 2)

    plsc.subcore_barrier()

    @pl.when(s < _NS - 1)
    def _():
        pltpu.sync_copy(acc.at[pl.ds(s * _NPT, _NPT)],
                        out_hbm.at[c, pl.ds(s * _NPT, _NPT)])

    @pl.when(s == _NS - 1)
    def _():
        pltpu.sync_copy(acc.at[pl.ds(s * _NPT, _NPL)],
                        out_hbm.at[c, pl.ds(s * _NPT, _NPL)])


def kernel(x, edge_index, weight):
    n, d = x.shape

    h, zeros = pl.pallas_call(
        _gemm_body,
        grid=(n // _BM,),
        in_specs=[pl.BlockSpec((_BM, d), lambda i: (i, 0)),
                  pl.BlockSpec(weight.shape, lambda i: (0, 0))],
        out_specs=[pl.BlockSpec((_BM, d), lambda i: (i, 0)),
                   pl.BlockSpec((_NZ, d), lambda i: (0, 0))],
        out_shape=[jax.ShapeDtypeStruct((n, d), jnp.float32),
                   jax.ShapeDtypeStruct((_NZ, d), jnp.float32)],
    )(x, weight)

    mesh = plsc.VectorSubcoreMesh(core_axis_name="c", subcore_axis_name="s")
    seg = pl.kernel(
        _seg_body,
        out_type=jax.ShapeDtypeStruct((_NC, n, d), jnp.float32),
        mesh=mesh,
        scratch_types=[
            pltpu.VMEM((2, _EPT), jnp.int32),
            pltpu.VMEM((3, _K, _D), jnp.float32),
            pltpu.SemaphoreType.DMA((3,)),
            pltpu.SemaphoreType.DMA((3,)),
            pltpu.VMEM_SHARED((_N, _D), jnp.float32),
        ],
    )
    parts = seg(edge_index, h, zeros)

    out = pl.pallas_call(
        _add_body,
        grid=(n // _BM,),
        in_specs=[pl.BlockSpec((_NC, _BM, d), lambda i: (0, i, 0))],
        out_specs=pl.BlockSpec((_BM, d), lambda i: (i, 0)),
        out_shape=jax.ShapeDtypeStruct((n, d), jnp.float32),
    )(parts)
    return out


# speedup vs baseline: 2.4277x; 1.2170x over previous
import jax
import jax.numpy as jnp
from jax import lax
from jax.experimental import pallas as pl
from jax.experimental.pallas import tpu as pltpu
from jax.experimental.pallas import tpu_sc as plsc

_N = 10000
_D = 128
_E = 320000
_NC = 2
_NS = 16
_NW = _NC * _NS
_K = 64
_EPT = 10112
_EPL = _E - 31 * _EPT
_CH = _EPT // _K
_CHL = _EPL // _K
_NPT = 632
_NPL = _N - (_NS - 1) * _NPT
_BM = 2000


def _gemm_body(x_ref, w_ref, o_ref, z_ref):
    o_ref[...] = lax.dot_general(
        x_ref[...], w_ref[...], (((1,), (1,)), ((), ())),
        preferred_element_type=jnp.float32)
    z_ref[...] = jnp.zeros_like(z_ref)


_NZ = 640


def _add_body(p_ref, o_ref):
    o_ref[...] = p_ref[0] + p_ref[1]


def _seg_body(e_hbm, h_hbm, z_hbm, out_hbm,
              eidx, rows, sems, ssems, acc):
    c = lax.axis_index("c")
    s = lax.axis_index("s")
    w = c * _NS + s
    nch = jnp.where(w < _NW - 1, _CH, _CHL)

    @pl.when(w < _NW - 1)
    def _():
        pltpu.sync_copy(e_hbm.at[:, pl.ds(w * _EPT, _EPT)], eidx)

    @pl.when(w == _NW - 1)
    def _():
        pltpu.sync_copy(e_hbm.at[:, pl.ds(w * _EPT, _EPL)],
                        eidx.at[:, pl.ds(0, _EPL)])
    @pl.when(s < _NS - 1)
    def _():
        pltpu.sync_copy(z_hbm.at[pl.ds(0, _NPT)],
                        acc.at[pl.ds(s * _NPT, _NPT)])

    @pl.when(s == _NS - 1)
    def _():
        pltpu.sync_copy(z_hbm.at[pl.ds(0, _NPL)],
                        acc.at[pl.ds(s * _NPT, _NPL)])

    plsc.subcore_barrier()

    def drain(ref_sems, b):
        pltpu.make_async_copy(h_hbm.at[pl.ds(0, _K)], rows.at[b],
                              ref_sems.at[b]).wait()

    def fire_gather(j, b):
        pltpu.async_copy(h_hbm.at[eidx.at[0, pl.ds(j * _K, _K)]],
                         rows.at[b], sems.at[b])

    fire_gather(0, 0)
    fire_gather(1, 1)

    def chunk(j, carry):
        b, bm1 = carry

        @pl.when(j > 0)
        def _():
            drain(ssems, bm1)

        @pl.when(j + 2 < nch)
        def _():
            fire_gather(j + 2, bm1)

        drain(sems, b)
        pltpu.async_copy(rows.at[b],
                         acc.at[eidx.at[1, pl.ds(j * _K, _K)]],
                         ssems.at[b], add=True)
        return (jnp.where(b == 2, 0, b + 1), b)

    lax.fori_loop(0, nch, chunk, (jnp.int32(0), jnp.int32(2)))
    drain(ssems, lax.rem(nch - 1, 3))

    plsc.subcore_barrier()

    @pl.when(s < _NS - 1)
    def _():
        pltpu.sync_copy(acc.at[pl.ds(s * _NPT, _NPT)],
                        out_hbm.at[c, pl.ds(s * _NPT, _NPT)])

    @pl.when(s == _NS - 1)
    def _():
        pltpu.sync_copy(acc.at[pl.ds(s * _NPT, _NPL)],
                        out_hbm.at[c, pl.ds(s * _NPT, _NPL)])


def kernel(x, edge_index, weight):
    n, d = x.shape

    h, zeros = pl.pallas_call(
        _gemm_body,
        grid=(n // _BM,),
        in_specs=[pl.BlockSpec((_BM, d), lambda i: (i, 0)),
                  pl.BlockSpec(weight.shape, lambda i: (0, 0))],
        out_specs=[pl.BlockSpec((_BM, d), lambda i: (i, 0)),
                   pl.BlockSpec((_NZ, d), lambda i: (0, 0))],
        out_shape=[jax.ShapeDtypeStruct((n, d), jnp.float32),
                   jax.ShapeDtypeStruct((_NZ, d), jnp.float32)],
    )(x, weight)

    mesh = plsc.VectorSubcoreMesh(core_axis_name="c", subcore_axis_name="s")
    seg = pl.kernel(
        _seg_body,
        out_type=jax.ShapeDtypeStruct((_NC, n, d), jnp.float32),
        mesh=mesh,
        scratch_types=[
            pltpu.VMEM((2, _EPT), jnp.int32),
            pltpu.VMEM((3, _K, _D), jnp.float32),
            pltpu.SemaphoreType.DMA((3,)),
            pltpu.SemaphoreType.DMA((3,)),
            pltpu.VMEM_SHARED((_N, _D), jnp.float32),
        ],
    )
    parts = seg(edge_index, h, zeros)

    out = pl.pallas_call(
        _add_body,
        grid=(n // _BM,),
        in_specs=[pl.BlockSpec((_NC, _BM, d), lambda i: (0, i, 0))],
        out_specs=pl.BlockSpec((_BM, d), lambda i: (i, 0)),
        out_shape=jax.ShapeDtypeStruct((n, d), jnp.float32),
    )(parts)
    return out
